# baseline (speedup 1.0000x reference)
import functools

import jax
import jax.numpy as jnp
from jax import lax
from jax.experimental import pallas as pl
from jax.experimental.pallas import tpu as pltpu
from jax.experimental.pallas import tpu_sc as plsc

_CHUNK = 128


@functools.cache
def _build(B, V, D, NC, NS):
    NW = NC * NS
    b_per_w = B // NW
    n_ch = b_per_w // _CHUNK
    mesh = plsc.VectorSubcoreMesh(core_axis_name="c", subcore_axis_name="s")

    @functools.partial(
        pl.kernel,
        mesh=mesh,
        out_type=jax.ShapeDtypeStruct((B, D), jnp.float32),
        scratch_types=[
            pltpu.VMEM((n_ch, _CHUNK), jnp.int32),
            pltpu.VMEM((b_per_w, D), jnp.float32),
            pltpu.SemaphoreType.DMA,
            pltpu.SemaphoreType.DMA,
        ],
    )
    def k(labels_hbm, table_hbm, out_hbm, idx_v, rows_v, gsem, wsem):
        wid = lax.axis_index("s") * NC + lax.axis_index("c")
        base = wid * b_per_w
        pltpu.sync_copy(labels_hbm.at[pl.ds(wid * n_ch, n_ch)], idx_v)
        gathers = []
        for j in range(n_ch):
            gathers.append(
                pltpu.async_copy(
                    table_hbm.at[idx_v.at[j]],
                    rows_v.at[pl.ds(j * _CHUNK, _CHUNK)],
                    gsem,
                )
            )
        writes = []
        for j in range(n_ch):
            gathers[j].wait()
            writes.append(
                pltpu.async_copy(
                    rows_v.at[pl.ds(j * _CHUNK, _CHUNK)],
                    out_hbm.at[pl.ds(base + j * _CHUNK, _CHUNK)],
                    wsem,
                )
            )
        for w in writes:
            w.wait()

    return k


def kernel(labels, table):
    (B,) = labels.shape
    V, D = table.shape
    info = plsc.get_sparse_core_info()
    labels2d = labels.astype(jnp.int32).reshape(B // _CHUNK, _CHUNK)
    return _build(B, V, D, info.num_cores, info.num_subcores)(labels2d, table)


# speedup vs baseline: 1.6176x; 1.6176x over previous
import functools

import jax
import jax.numpy as jnp
from jax import lax
from jax.experimental import pallas as pl
from jax.experimental.pallas import tpu as pltpu
from jax.experimental.pallas import tpu_sc as plsc

_CHUNK = 128


@functools.cache
def _build(B, V, D, NC, NS):
    NW = NC * NS
    b_per_w = B // NW
    n_ch = b_per_w // _CHUNK
    mesh = plsc.VectorSubcoreMesh(core_axis_name="c", subcore_axis_name="s")

    @functools.partial(
        pl.kernel,
        mesh=mesh,
        out_type=jax.ShapeDtypeStruct((B, D), jnp.float32),
        scratch_types=[
            pltpu.VMEM((n_ch, _CHUNK), jnp.int32),
            pltpu.VMEM((b_per_w, D), jnp.float32),
            pltpu.VMEM_SHARED((V, D), jnp.float32),
            pltpu.SemaphoreType.DMA,
        ],
    )
    def k(labels_hbm, table_hbm, out_hbm, idx_v, rows_v, table_sh, gsem):
        cid = lax.axis_index("c")
        sid = lax.axis_index("s")
        wid = sid * NC + cid
        base = wid * b_per_w

        @pl.when(sid == 0)
        def _():
            pltpu.sync_copy(table_hbm, rows_v.at[pl.ds(0, V)])
            pltpu.sync_copy(rows_v.at[pl.ds(0, V)], table_sh)

        pltpu.sync_copy(labels_hbm.at[pl.ds(wid * n_ch, n_ch)], idx_v)
        plsc.subcore_barrier()

        gathers = []
        for j in range(n_ch):
            gathers.append(
                pltpu.async_copy(
                    table_sh.at[idx_v.at[j]],
                    rows_v.at[pl.ds(j * _CHUNK, _CHUNK)],
                    gsem,
                )
            )
        for g in gathers:
            g.wait()
        pltpu.sync_copy(rows_v, out_hbm.at[pl.ds(base, b_per_w)])

    return k


def kernel(labels, table):
    (B,) = labels.shape
    V, D = table.shape
    info = plsc.get_sparse_core_info()
    labels2d = labels.astype(jnp.int32).reshape(B // _CHUNK, _CHUNK)
    return _build(B, V, D, info.num_cores, info.num_subcores)(labels2d, table)


# speedup vs baseline: 1.6914x; 1.0456x over previous
import functools

import jax
import jax.numpy as jnp
from jax import lax
from jax.experimental import pallas as pl
from jax.experimental.pallas import tpu as pltpu
from jax.experimental.pallas import tpu_sc as plsc

_CHUNK = 128


@functools.cache
def _build(B, V, D, NC, NS):
    NW = NC * NS
    b_per_w = B // NW
    n_ch = b_per_w // _CHUNK
    mesh = plsc.VectorSubcoreMesh(core_axis_name="c", subcore_axis_name="s")

    @functools.partial(
        pl.kernel,
        mesh=mesh,
        out_type=jax.ShapeDtypeStruct((B, D), jnp.float32),
        scratch_types=[
            pltpu.VMEM((n_ch, _CHUNK), jnp.int32),
            pltpu.VMEM((b_per_w, D), jnp.float32),
            pltpu.VMEM_SHARED((V, D), jnp.float32),
            pltpu.SemaphoreType.DMA,
            pltpu.SemaphoreType.DMA,
        ],
    )
    def k(labels_hbm, table_hbm, out_hbm, idx_v, rows_v, table_sh, gsem, wsem):
        cid = lax.axis_index("c")
        sid = lax.axis_index("s")
        wid = sid * NC + cid
        base = wid * b_per_w

        @pl.when(sid == 0)
        def _():
            pltpu.sync_copy(table_hbm, rows_v.at[pl.ds(0, V)])
            pltpu.sync_copy(rows_v.at[pl.ds(0, V)], table_sh)

        pltpu.sync_copy(labels_hbm.at[pl.ds(wid * n_ch, n_ch)], idx_v)
        plsc.subcore_barrier()

        gathers = []
        for j in range(n_ch):
            gathers.append(
                pltpu.async_copy(
                    table_sh.at[idx_v.at[j]],
                    rows_v.at[pl.ds(j * _CHUNK, _CHUNK)],
                    gsem,
                )
            )
        writes = []
        for j in range(n_ch):
            gathers[j].wait()
            writes.append(
                pltpu.async_copy(
                    rows_v.at[pl.ds(j * _CHUNK, _CHUNK)],
                    out_hbm.at[pl.ds(base + j * _CHUNK, _CHUNK)],
                    wsem,
                )
            )
        for w in writes:
            w.wait()

    return k


def kernel(labels, table):
    (B,) = labels.shape
    V, D = table.shape
    info = plsc.get_sparse_core_info()
    labels2d = labels.astype(jnp.int32).reshape(B // _CHUNK, _CHUNK)
    return _build(B, V, D, info.num_cores, info.num_subcores)(labels2d, table)


# speedup vs baseline: 1.7391x; 1.0282x over previous
import functools

import jax
import jax.numpy as jnp
from jax import lax
from jax.experimental import pallas as pl
from jax.experimental.pallas import tpu as pltpu
from jax.experimental.pallas import tpu_sc as plsc

_CHUNK = 128


@functools.cache
def _build(B, V, D, NC, NS):
    NW = NC * NS
    b_per_w = B // NW
    n_ch = b_per_w // _CHUNK
    V_pad = -(-V // (8 * NS)) * (8 * NS)
    mesh = plsc.VectorSubcoreMesh(core_axis_name="c", subcore_axis_name="s")

    @functools.partial(
        pl.kernel,
        mesh=mesh,
        out_type=jax.ShapeDtypeStruct((B, D), jnp.float32),
        scratch_types=[
            pltpu.VMEM((n_ch, _CHUNK), jnp.int32),
            pltpu.VMEM((b_per_w, D), jnp.float32),
            pltpu.VMEM_SHARED((V_pad, D), jnp.float32),
            pltpu.SemaphoreType.DMA,
            pltpu.SemaphoreType.DMA,
        ],
    )
    def k(labels_hbm, table_hbm, out_hbm, idx_v, rows_v, table_sh, gsem, wsem):
        cid = lax.axis_index("c")
        sid = lax.axis_index("s")
        wid = sid * NC + cid
        base = wid * b_per_w

        rpt = table_sh.shape[0] // NS
        pltpu.sync_copy(
            table_hbm.at[pl.ds(sid * rpt, rpt)],
            table_sh.at[pl.ds(sid * rpt, rpt)],
        )

        pltpu.sync_copy(labels_hbm.at[pl.ds(wid * n_ch, n_ch)], idx_v)
        plsc.subcore_barrier()

        gathers = []
        for j in range(n_ch):
            gathers.append(
                pltpu.async_copy(
                    table_sh.at[idx_v.at[j]],
                    rows_v.at[pl.ds(j * _CHUNK, _CHUNK)],
                    gsem,
                )
            )
        writes = []
        for j in range(n_ch):
            gathers[j].wait()
            writes.append(
                pltpu.async_copy(
                    rows_v.at[pl.ds(j * _CHUNK, _CHUNK)],
                    out_hbm.at[pl.ds(base + j * _CHUNK, _CHUNK)],
                    wsem,
                )
            )
        for w in writes:
            w.wait()

    return k


def kernel(labels, table):
    (B,) = labels.shape
    V, D = table.shape
    info = plsc.get_sparse_core_info()
    NS = info.num_subcores
    V_pad = -(-V // (8 * NS)) * (8 * NS)
    labels2d = labels.astype(jnp.int32).reshape(B // _CHUNK, _CHUNK)
    table_p = jnp.pad(table, ((0, V_pad - V), (0, 0)))
    return _build(B, V, D, info.num_cores, NS)(labels2d, table_p)
